# baseline (speedup 1.0000x reference)
import functools

import jax
import jax.numpy as jnp
from jax import lax
from jax.experimental import pallas as pl
from jax.experimental.pallas import tpu as pltpu
from jax.experimental.pallas import tpu_sc as plsc

NUM_CORES = 2
NUM_SUBCORES = 16
NW = NUM_CORES * NUM_SUBCORES
BATCH = 16384
BPW = BATCH // NW
MF_DIM = 32
IDX_CHUNK = 128
NCHUNK = BPW // IDX_CHUNK
GROUPS = BPW // 16

_mesh = plsc.VectorSubcoreMesh(core_axis_name="c", subcore_axis_name="s")


@functools.partial(
    pl.kernel,
    mesh=_mesh,
    compiler_params=pltpu.CompilerParams(
        needs_layout_passes=False, use_tc_tiling_on_sc=False),
    out_type=(
        jax.ShapeDtypeStruct((BATCH,), jnp.float32),
        jax.ShapeDtypeStruct((BATCH,), jnp.float32),
        jax.ShapeDtypeStruct((NW * 16,), jnp.float32),
    ),
    scratch_types=[
        pltpu.VMEM((BPW,), jnp.int32),
        pltpu.VMEM((BPW,), jnp.int32),
        pltpu.VMEM((BPW,), jnp.float32),
        pltpu.VMEM((MF_DIM, BPW), jnp.float32),
        pltpu.VMEM((MF_DIM, BPW), jnp.float32),
        pltpu.VMEM((BPW,), jnp.float32),
        pltpu.VMEM((BPW,), jnp.float32),
        pltpu.VMEM((16,), jnp.float32),
        pltpu.VMEM((BPW,), jnp.float32),
        pltpu.VMEM((BPW,), jnp.float32),
        pltpu.VMEM((16,), jnp.float32),
        pltpu.SemaphoreType.DMA,
    ],
)
def _pmf_sc(user_hbm, item_hbm, label_hbm, utabT_hbm, itabT_hbm,
            ubias_hbm, ibias_hbm, avg_hbm,
            pred_hbm, loss_hbm, obj_hbm,
            uidx_v, iidx_v, lbl_v, urowsT_v, irowsT_v, ub_v, ib_v,
            avg_v, pred_v, loss_v, obj_v, sem):
    wid = lax.axis_index("s") * NUM_CORES + lax.axis_index("c")
    base = wid * BPW

    pltpu.sync_copy(user_hbm.at[pl.ds(base, BPW)], uidx_v)
    pltpu.sync_copy(item_hbm.at[pl.ds(base, BPW)], iidx_v)
    pltpu.sync_copy(label_hbm.at[pl.ds(base, BPW)], lbl_v)
    pltpu.sync_copy(avg_hbm, avg_v)

    copies = []
    for ch in range(NCHUNK):
        sl = pl.ds(ch * IDX_CHUNK, IDX_CHUNK)
        for j in range(MF_DIM):
            copies.append(pltpu.make_async_copy(
                utabT_hbm.at[j].at[uidx_v.at[sl]], urowsT_v.at[j, sl], sem))
            copies.append(pltpu.make_async_copy(
                itabT_hbm.at[j].at[iidx_v.at[sl]], irowsT_v.at[j, sl], sem))
        copies.append(pltpu.make_async_copy(
            ubias_hbm.at[uidx_v.at[sl]], ub_v.at[sl], sem))
        copies.append(pltpu.make_async_copy(
            ibias_hbm.at[iidx_v.at[sl]], ib_v.at[sl], sem))
    for cp in copies:
        cp.start()
    for cp in copies:
        cp.wait()

    avg16 = avg_v[...]

    def group_body(g, obj_acc):
        off = g * 16
        sl16 = pl.ds(off, 16)
        acc = jnp.zeros((16,), jnp.float32)
        for j in range(MF_DIM):
            acc = acc + urowsT_v[j, sl16] * irowsT_v[j, sl16]
        pred = acc + avg16 + ub_v[sl16] + ib_v[sl16]
        diff = pred - lbl_v[sl16]
        loss = diff * diff
        pred_v[sl16] = pred
        loss_v[sl16] = loss
        return obj_acc + loss

    obj16 = lax.fori_loop(0, GROUPS, group_body, jnp.zeros((16,), jnp.float32))
    obj_v[...] = obj16

    pltpu.sync_copy(pred_v, pred_hbm.at[pl.ds(base, BPW)])
    pltpu.sync_copy(loss_v, loss_hbm.at[pl.ds(base, BPW)])
    pltpu.sync_copy(obj_v, obj_hbm.at[pl.ds(wid * 16, 16)])


def kernel(user, item, label, user_table, item_table, user_bias, item_bias, avg_rating):
    ubias_f = user_bias.reshape(-1)
    ibias_f = item_bias.reshape(-1)
    avg16 = jnp.broadcast_to(jnp.asarray(avg_rating, jnp.float32).reshape(1), (16,))
    utabT = jnp.swapaxes(user_table, 0, 1)
    itabT = jnp.swapaxes(item_table, 0, 1)
    pred, loss, obj_part = _pmf_sc(user.astype(jnp.int32), item.astype(jnp.int32),
                                   label, utabT, itabT,
                                   ubias_f, ibias_f, avg16)
    return (pred, loss, jnp.sum(obj_part))


# speedup vs baseline: 5.6956x; 5.6956x over previous
import functools

import jax
import jax.numpy as jnp
from jax import lax
from jax.experimental import pallas as pl
from jax.experimental.pallas import tpu as pltpu
from jax.experimental.pallas import tpu_sc as plsc

NUM_CORES = 2
NUM_SUBCORES = 16
NW = NUM_CORES * NUM_SUBCORES
BATCH = 16384
BPW = BATCH // NW
MF_DIM = 32
IDX_CHUNK = 128
NCHUNK = BPW // IDX_CHUNK
GROUPS = BPW // 16

_mesh = plsc.VectorSubcoreMesh(core_axis_name="c", subcore_axis_name="s")


@functools.partial(
    pl.kernel,
    mesh=_mesh,
    compiler_params=pltpu.CompilerParams(
        needs_layout_passes=False, use_tc_tiling_on_sc=False),
    out_type=(
        jax.ShapeDtypeStruct((BATCH,), jnp.float32),
        jax.ShapeDtypeStruct((BATCH,), jnp.float32),
        jax.ShapeDtypeStruct((NW * 16,), jnp.float32),
    ),
    scratch_types=[
        pltpu.VMEM((BPW,), jnp.int32),
        pltpu.VMEM((BPW,), jnp.int32),
        pltpu.VMEM((BPW,), jnp.float32),
        pltpu.VMEM((BPW, MF_DIM), jnp.float32),
        pltpu.VMEM((BPW, MF_DIM), jnp.float32),
        pltpu.VMEM((BPW,), jnp.float32),
        pltpu.VMEM((BPW,), jnp.float32),
        pltpu.VMEM((16,), jnp.float32),
        pltpu.VMEM((BPW,), jnp.float32),
        pltpu.VMEM((BPW,), jnp.float32),
        pltpu.VMEM((16,), jnp.float32),
        pltpu.SemaphoreType.DMA,
    ],
)
def _pmf_sc(user_hbm, item_hbm, label_hbm, utab_hbm, itab_hbm,
            ubias_hbm, ibias_hbm, avg_hbm,
            pred_hbm, loss_hbm, obj_hbm,
            uidx_v, iidx_v, lbl_v, urows_v, irows_v, ub_v, ib_v,
            avg_v, pred_v, loss_v, obj_v, sem):
    wid = lax.axis_index("s") * NUM_CORES + lax.axis_index("c")
    base = wid * BPW

    pltpu.sync_copy(user_hbm.at[pl.ds(base, BPW)], uidx_v)
    pltpu.sync_copy(item_hbm.at[pl.ds(base, BPW)], iidx_v)
    pltpu.sync_copy(label_hbm.at[pl.ds(base, BPW)], lbl_v)
    pltpu.sync_copy(avg_hbm, avg_v)

    copies = []
    for ch in range(NCHUNK):
        sl = pl.ds(ch * IDX_CHUNK, IDX_CHUNK)
        copies.append(pltpu.make_async_copy(
            utab_hbm.at[uidx_v.at[sl]], urows_v.at[sl], sem))
        copies.append(pltpu.make_async_copy(
            itab_hbm.at[iidx_v.at[sl]], irows_v.at[sl], sem))
        copies.append(pltpu.make_async_copy(
            ubias_hbm.at[uidx_v.at[sl]], ub_v.at[sl], sem))
        copies.append(pltpu.make_async_copy(
            ibias_hbm.at[iidx_v.at[sl]], ib_v.at[sl], sem))
    for cp in copies:
        cp.start()
    for cp in copies:
        cp.wait()

    avg16 = avg_v[...]
    lane = lax.iota(jnp.int32, 16)
    cols = [jnp.full((16,), j, jnp.int32) for j in range(MF_DIM)]

    def group_body(g, obj_acc):
        rows = g * 16 + lane
        acc = jnp.zeros((16,), jnp.float32)
        for j in range(MF_DIM):
            uj = plsc.load_gather(urows_v, [rows, cols[j]])
            vj = plsc.load_gather(irows_v, [rows, cols[j]])
            acc = acc + uj * vj
        off = g * 16
        pred = acc + avg16 + ub_v[pl.ds(off, 16)] + ib_v[pl.ds(off, 16)]
        diff = pred - lbl_v[pl.ds(off, 16)]
        loss = diff * diff
        pred_v[pl.ds(off, 16)] = pred
        loss_v[pl.ds(off, 16)] = loss
        return obj_acc + loss

    obj16 = lax.fori_loop(0, GROUPS, group_body, jnp.zeros((16,), jnp.float32))
    obj_v[...] = obj16

    pltpu.sync_copy(pred_v, pred_hbm.at[pl.ds(base, BPW)])
    pltpu.sync_copy(loss_v, loss_hbm.at[pl.ds(base, BPW)])
    pltpu.sync_copy(obj_v, obj_hbm.at[pl.ds(wid * 16, 16)])


def kernel(user, item, label, user_table, item_table, user_bias, item_bias, avg_rating):
    ubias_f = user_bias.reshape(-1)
    ibias_f = item_bias.reshape(-1)
    avg16 = jnp.broadcast_to(jnp.asarray(avg_rating, jnp.float32).reshape(1), (16,))
    utab = jnp.swapaxes(lax.optimization_barrier(
        jnp.swapaxes(user_table, 0, 1)), 0, 1)
    itab = jnp.swapaxes(lax.optimization_barrier(
        jnp.swapaxes(item_table, 0, 1)), 0, 1)
    pred, loss, obj_part = _pmf_sc(user.astype(jnp.int32), item.astype(jnp.int32),
                                   label, utab, itab,
                                   ubias_f, ibias_f, avg16)
    return (pred, loss, jnp.sum(obj_part))


# speedup vs baseline: 16.0992x; 2.8266x over previous
import functools

import jax
import jax.numpy as jnp
from jax import lax
from jax.experimental import pallas as pl
from jax.experimental.pallas import tpu as pltpu
from jax.experimental.pallas import tpu_sc as plsc

NUM_CORES = 2
NUM_SUBCORES = 16
NW = NUM_CORES * NUM_SUBCORES
BATCH = 16384
BPW = BATCH // NW
MF_DIM = 32
NSEG = MF_DIM // 8
PITCH = 1 << 20
WBLK = PITCH // 8
IDX_CHUNK = 128
NCHUNK = BPW // IDX_CHUNK
GROUPS = BPW // 16

_mesh = plsc.VectorSubcoreMesh(core_axis_name="c", subcore_axis_name="s")


def _detile_body(in_ref, *out_refs):
    for s in range(8):
        out_refs[s][...] = in_ref[s, :]


def _detile(tabT):
    nblk = PITCH // WBLK
    return pl.pallas_call(
        _detile_body,
        grid=(NSEG, nblk),
        in_specs=[pl.BlockSpec((8, WBLK), lambda t, c: (t, c))],
        out_specs=[pl.BlockSpec((WBLK,), lambda t, c: (t * nblk + c))
                   for _ in range(8)],
        out_shape=[jax.ShapeDtypeStruct((NSEG * PITCH,), jnp.float32)
                   for _ in range(8)],
    )(tabT)


@functools.partial(
    pl.kernel,
    mesh=_mesh,
    compiler_params=pltpu.CompilerParams(
        needs_layout_passes=False, use_tc_tiling_on_sc=False),
    out_type=(
        jax.ShapeDtypeStruct((BATCH,), jnp.float32),
        jax.ShapeDtypeStruct((BATCH,), jnp.float32),
        jax.ShapeDtypeStruct((NW * 16,), jnp.float32),
    ),
    scratch_types=[
        pltpu.VMEM((BPW,), jnp.int32),
        pltpu.VMEM((BPW,), jnp.int32),
        pltpu.VMEM((BPW,), jnp.float32),
        pltpu.VMEM((NSEG, IDX_CHUNK), jnp.int32),
        pltpu.VMEM((NSEG, IDX_CHUNK), jnp.int32),
        pltpu.VMEM((MF_DIM, BPW), jnp.float32),
        pltpu.VMEM((MF_DIM, BPW), jnp.float32),
        pltpu.VMEM((BPW,), jnp.float32),
        pltpu.VMEM((BPW,), jnp.float32),
        pltpu.VMEM((16,), jnp.float32),
        pltpu.VMEM((BPW,), jnp.float32),
        pltpu.VMEM((BPW,), jnp.float32),
        pltpu.VMEM((16,), jnp.float32),
        pltpu.SemaphoreType.DMA,
    ],
)
def _pmf_sc(user_hbm, item_hbm, label_hbm,
            ut0, ut1, ut2, ut3, ut4, ut5, ut6, ut7,
            it0, it1, it2, it3, it4, it5, it6, it7,
            ubias_hbm, ibias_hbm, avg_hbm,
            pred_hbm, loss_hbm, obj_hbm,
            uidx_v, iidx_v, lbl_v, gidx_u, gidx_i, urowsT_v, irowsT_v,
            ub_v, ib_v, avg_v, pred_v, loss_v, obj_v, sem):
    utabs = [ut0, ut1, ut2, ut3, ut4, ut5, ut6, ut7]
    itabs = [it0, it1, it2, it3, it4, it5, it6, it7]
    wid = lax.axis_index("s") * NUM_CORES + lax.axis_index("c")
    base = wid * BPW

    pltpu.sync_copy(user_hbm.at[pl.ds(base, BPW)], uidx_v)
    pltpu.sync_copy(item_hbm.at[pl.ds(base, BPW)], iidx_v)
    pltpu.sync_copy(label_hbm.at[pl.ds(base, BPW)], lbl_v)
    pltpu.sync_copy(avg_hbm, avg_v)

    for ch in range(NCHUNK):
        sl = pl.ds(ch * IDX_CHUNK, IDX_CHUNK)
        for p in range(IDX_CHUNK // 16):
            uv = uidx_v[pl.ds(ch * IDX_CHUNK + p * 16, 16)]
            iv = iidx_v[pl.ds(ch * IDX_CHUNK + p * 16, 16)]
            for t in range(NSEG):
                gidx_u[t, pl.ds(p * 16, 16)] = uv + t * PITCH
                gidx_i[t, pl.ds(p * 16, 16)] = iv + t * PITCH
        wave = []
        for j in range(MF_DIM):
            t, s = j // 8, j % 8
            wave.append(pltpu.make_async_copy(
                utabs[s].at[gidx_u.at[t]], urowsT_v.at[j, sl], sem))
            wave.append(pltpu.make_async_copy(
                itabs[s].at[gidx_i.at[t]], irowsT_v.at[j, sl], sem))
        wave.append(pltpu.make_async_copy(
            ubias_hbm.at[uidx_v.at[sl]], ub_v.at[sl], sem))
        wave.append(pltpu.make_async_copy(
            ibias_hbm.at[iidx_v.at[sl]], ib_v.at[sl], sem))
        for cp in wave:
            cp.start()
        for cp in wave:
            cp.wait()

    avg16 = avg_v[...]

    def group_body(g, obj_acc):
        sl16 = pl.ds(g * 16, 16)
        acc = jnp.zeros((16,), jnp.float32)
        for j in range(MF_DIM):
            acc = acc + urowsT_v[j, sl16] * irowsT_v[j, sl16]
        pred = acc + avg16 + ub_v[sl16] + ib_v[sl16]
        diff = pred - lbl_v[sl16]
        loss = diff * diff
        pred_v[sl16] = pred
        loss_v[sl16] = loss
        return obj_acc + loss

    obj16 = lax.fori_loop(0, GROUPS, group_body, jnp.zeros((16,), jnp.float32))
    obj_v[...] = obj16

    pltpu.sync_copy(pred_v, pred_hbm.at[pl.ds(base, BPW)])
    pltpu.sync_copy(loss_v, loss_hbm.at[pl.ds(base, BPW)])
    pltpu.sync_copy(obj_v, obj_hbm.at[pl.ds(wid * 16, 16)])


def kernel(user, item, label, user_table, item_table, user_bias, item_bias, avg_rating):
    ubias_f = user_bias.reshape(-1)
    ibias_f = item_bias.reshape(-1)
    avg16 = jnp.broadcast_to(jnp.asarray(avg_rating, jnp.float32).reshape(1), (16,))
    utabs = _detile(jnp.swapaxes(user_table, 0, 1))
    itabs = _detile(jnp.swapaxes(item_table, 0, 1))
    pred, loss, obj_part = _pmf_sc(user.astype(jnp.int32), item.astype(jnp.int32),
                                   label, *utabs, *itabs,
                                   ubias_f, ibias_f, avg16)
    return (pred, loss, jnp.sum(obj_part))
